# baseline (speedup 1.0000x reference)
import jax
import jax.numpy as jnp
from jax import lax
from jax.experimental import pallas as pl
from jax.experimental.pallas import tpu as pltpu
from jax.experimental.pallas import tpu_sc as plsc

_N = 22
_NP = 32
_E = 484
_EP = 512

_f32 = jnp.float32
_i32 = jnp.int32


def _iota16():
    return lax.broadcasted_iota(_i32, (16,), 0)


def _splat(v):
    return jnp.full((16,), v, _i32)


def _sc_body(x_h, srcw_h, dstw_h, wl_h, bl_h, w1_h, b1_h, w2_h, b2_h, wr_h,
             br_h, out_h,
             x_v, srcw_v, dstw_v, wl_v, bl_v, w1_v, b1_v, w2_v, b2_v,
             wr_v, br_v, h_v, agg_v, msg_v, zeros_v, h_sh, agg_sh, outv_v,
             sem):
    cid = lax.axis_index("c")
    sid = lax.axis_index("s")
    is0 = jnp.logical_and(cid == 0, sid == 0)

    @pl.when(is0)
    def _work():
        copies = [pltpu.async_copy(s, d, sem) for s, d in (
            (x_h, x_v), (srcw_h, srcw_v),
            (dstw_h, dstw_v), (wl_h, wl_v), (bl_h, bl_v),
            (w1_h, w1_v), (b1_h, b1_v), (w2_h, w2_v), (b2_h, b2_v),
            (wr_h, wr_v), (br_h, br_v))]
        iot = _iota16()
        zero16 = jnp.zeros((16,), _f32)
        for r in range(_NP):
            zeros_v[pl.ds(r * 16, 16)] = zero16
        for c in copies:
            c.wait()

        def wrow(ref, k, width):
            return plsc.load_gather(ref, [_splat(k * width)
                                          + jnp.minimum(iot, width - 1)])

        def brow(ref, width):
            return plsc.load_gather(ref, [jnp.minimum(iot, width - 1)])

        def dense_from_agg(in_f, w_rows, b_row):
            def nbody(n, carry):
                acc = b_row
                for k in range(in_f):
                    g = plsc.load_gather(agg_v, [_splat(n * 16 + k)])
                    acc = acc + g * w_rows[k]
                plsc.store_scatter(h_v, [_splat(n * 16) + iot],
                                   jnp.maximum(acc, 0.0))
                return carry
            lax.fori_loop(0, _N, nbody, 0)

        def edge_stage(n_feat):
            pltpu.sync_copy(h_v, h_sh)
            pltpu.sync_copy(zeros_v, agg_sh)
            pltpu.async_copy(h_sh.at[srcw_v], msg_v, sem).wait()
            pltpu.sync_copy(msg_v, agg_sh.at[dstw_v], add=True)
            pltpu.sync_copy(agg_sh, agg_v)

        wl_rows = [wrow(wl_v, k, 15) for k in range(9)]
        bl_row = brow(bl_v, 15)

        def lift_body(n, carry):
            acc = bl_row
            for k in range(9):
                g = plsc.load_gather(x_v, [_splat(n * 9 + k)])
                acc = acc + g * wl_rows[k]
            plsc.store_scatter(h_v, [_splat(n * 16) + iot],
                               jnp.maximum(acc, 0.0))
            return carry
        lax.fori_loop(0, _N, lift_body, 0)

        edge_stage(15)
        w1_rows = [wrow(w1_v, k, 10) for k in range(15)]
        dense_from_agg(15, w1_rows, brow(b1_v, 10))
        edge_stage(10)
        w2_rows = [wrow(w2_v, k, 5) for k in range(10)]
        dense_from_agg(10, w2_rows, brow(b2_v, 5))

        def robody(n, acc):
            hrw = plsc.load_gather(h_v, [_splat(n * 16) + iot])
            ridx = jnp.minimum(_splat(n * 5) + iot, _splat(109))
            wrw = plsc.load_gather(wr_v, [ridx])
            return acc + jnp.where(iot < 5, hrw * wrw, 0.0)

        acc = lax.fori_loop(0, _N, robody, jnp.zeros((16,), _f32))
        total = jnp.sum(acc)
        outv_v[...] = plsc.load_gather(br_v, [_splat(0)]) + total
        pltpu.sync_copy(outv_v.at[pl.ds(0, 1)], out_h)


@jax.jit
def _sc_call(x, edge_index, W_lift, b_lift, W1, b1, W2, b2, Wr, br):
    mesh = plsc.VectorSubcoreMesh(core_axis_name="c", subcore_axis_name="s",
                                  num_cores=2, num_subcores=16)
    f = pl.kernel(
        _sc_body,
        out_type=jax.ShapeDtypeStruct((1,), _f32),
        mesh=mesh,
        compiler_params=pltpu.CompilerParams(needs_layout_passes=False),
        scratch_types=[
            pltpu.VMEM((_N * 9,), _f32),
            pltpu.VMEM((_EP * 16,), _i32),
            pltpu.VMEM((_EP * 16,), _i32),
            pltpu.VMEM((9 * 15,), _f32),
            pltpu.VMEM((15,), _f32),
            pltpu.VMEM((15 * 10,), _f32),
            pltpu.VMEM((10,), _f32),
            pltpu.VMEM((10 * 5,), _f32),
            pltpu.VMEM((5,), _f32),
            pltpu.VMEM((110,), _f32),
            pltpu.VMEM((1,), _f32),
            pltpu.VMEM((_NP * 16,), _f32),
            pltpu.VMEM((_NP * 16,), _f32),
            pltpu.VMEM((_EP * 16,), _f32),
            pltpu.VMEM((_NP * 16,), _f32),
            pltpu.VMEM_SHARED((_NP * 16,), _f32),
            pltpu.VMEM_SHARED((_NP * 16,), _f32),
            pltpu.VMEM((16,), _f32),
            pltpu.SemaphoreType.DMA,
        ],
    )
    lane = jnp.arange(16, dtype=_i32)
    srcp = jnp.concatenate([edge_index[0],
                            jnp.zeros((_EP - _E,), _i32)])
    dstp = jnp.concatenate([edge_index[1],
                            jnp.full((_EP - _E,), _NP - 1, _i32)])
    srcw = (srcp[:, None] * 16 + lane).reshape(-1)
    dstw = (dstp[:, None] * 16 + lane).reshape(-1)
    out = f(x.reshape(-1), srcw, dstw, W_lift.reshape(-1),
            b_lift, W1.reshape(-1), b1, W2.reshape(-1), b2,
            Wr.reshape(-1), br)
    return out.reshape(1, 1)


def _tc_body(src_ref, dst_ref, x_ref, wl_ref, bl_ref, w1_ref, b1_ref,
             w2_ref, b2_ref, wr_ref, br_ref, out_ref):
    f32 = jnp.float32
    nodes = jax.lax.broadcasted_iota(jnp.int32, (_N, _E), 0)
    d_oh = (dst_ref[...] == nodes).astype(f32)
    s_oh = (src_ref[...] == nodes).astype(f32)
    adj = jax.lax.dot_general(d_oh, s_oh, (((1,), (1,)), ((), ())),
                              preferred_element_type=f32)
    h = jnp.maximum(
        jnp.dot(x_ref[...], wl_ref[...], preferred_element_type=f32)
        + bl_ref[...], 0.0)
    agg = jnp.dot(adj, h, preferred_element_type=f32)
    h = jnp.maximum(
        jnp.dot(agg, w1_ref[...], preferred_element_type=f32)
        + b1_ref[...], 0.0)
    agg = jnp.dot(adj, h, preferred_element_type=f32)
    h = jnp.maximum(
        jnp.dot(agg, w2_ref[...], preferred_element_type=f32)
        + b2_ref[...], 0.0)
    out_ref[...] = jnp.sum(h * wr_ref[...])[None, None] + br_ref[...]


def _tc_call(x, edge_index, W_lift, b_lift, W1, b1, W2, b2, Wr, br):
    src = edge_index[0].reshape(1, _E)
    dst = edge_index[1].reshape(1, _E)
    out = pl.pallas_call(
        _tc_body,
        out_shape=jax.ShapeDtypeStruct((1, 1), jnp.float32),
    )(src, dst, x, W_lift, b_lift.reshape(1, -1), W1, b1.reshape(1, -1),
      W2, b2.reshape(1, -1), Wr.reshape(_N, 5), br.reshape(1, 1))
    return out


kernel = _sc_call


# speedup vs baseline: 5.1605x; 5.1605x over previous
import jax
import jax.numpy as jnp
from jax import lax
from jax.experimental import pallas as pl
from jax.experimental.pallas import tpu as pltpu
from jax.experimental.pallas import tpu_sc as plsc

_N = 22
_NP = 32
_E = 484
_EP = 512
_F = 16

_f32 = jnp.float32
_i32 = jnp.int32


def _iota16():
    return lax.broadcasted_iota(_i32, (16,), 0)


def _splat(v):
    return jnp.full((16,), v, _i32)


def _sc_body(x_h, srcw_h, dstw_h, wl_h, bl_h, w1_h, b1_h, w2_h, b2_h, wr_h,
             br_h, out_h,
             x_v, srcw_v, dstw_v, wl_v, bl_v, w1_v, b1_v, w2_v, b2_v,
             wr_v, br_v, h_v, agg_v, msg_v, zeros_v, h_sh, agg_sh, outv_v,
             sem):
    cid = lax.axis_index("c")
    sid = lax.axis_index("s")
    is0 = jnp.logical_and(cid == 0, sid == 0)

    @pl.when(is0)
    def _work():
        _sc_work(x_h, srcw_h, dstw_h, wl_h, bl_h, w1_h, b1_h, w2_h, b2_h,
                 wr_h, br_h, out_h,
                 x_v, srcw_v, dstw_v, wl_v, bl_v, w1_v, b1_v, w2_v, b2_v,
                 wr_v, br_v, h_v, agg_v, msg_v, zeros_v, h_sh, agg_sh,
                 outv_v, sem)


def _sc_work(x_h, srcw_h, dstw_h, wl_h, bl_h, w1_h, b1_h, w2_h, b2_h, wr_h,
             br_h, out_h,
             x_v, srcw_v, dstw_v, wl_v, bl_v, w1_v, b1_v, w2_v, b2_v,
             wr_v, br_v, h_v, agg_v, msg_v, zeros_v, h_sh, agg_sh, outv_v,
             sem):
    copies = [pltpu.async_copy(s, d, sem) for s, d in (
        (x_h, x_v), (srcw_h, srcw_v),
        (dstw_h, dstw_v), (wl_h, wl_v), (bl_h, bl_v),
        (w1_h, w1_v), (b1_h, b1_v), (w2_h, w2_v), (b2_h, b2_v),
        (wr_h, wr_v), (br_h, br_v))]
    zero16 = jnp.zeros((16,), _f32)
    for r in range(_F * _NP // 16):
        zeros_v[pl.ds(r * 16, 16)] = zero16
    h_v[pl.ds(15 * _NP, 16)] = zero16
    h_v[pl.ds(15 * _NP + 16, 16)] = zero16
    for c in copies:
        c.wait()

    def wsplat(ref, idx):
        return plsc.load_gather(ref, [_splat(idx)])

    def edge_stage():
        pltpu.sync_copy(h_v, h_sh)
        pltpu.sync_copy(zeros_v, agg_sh)
        pltpu.async_copy(h_sh.at[srcw_v], msg_v, sem).wait()
        pltpu.sync_copy(msg_v, agg_sh.at[dstw_v], add=True)
        pltpu.sync_copy(agg_sh, agg_v)

    def dense(in_ref, out_ref, w_ref, b_ref, n_in, n_out):
        ins = [(in_ref[pl.ds(k * _NP, 16)],
                in_ref[pl.ds(k * _NP + 16, 16)])
               for k in range(n_in)]
        for j in range(n_out):
            b = wsplat(b_ref, j)
            acc0 = b
            acc1 = b
            for k in range(n_in):
                w = wsplat(w_ref, k * n_out + j)
                acc0 = acc0 + ins[k][0] * w
                acc1 = acc1 + ins[k][1] * w
            out_ref[pl.ds(j * _NP, 16)] = jnp.maximum(acc0, 0.0)
            out_ref[pl.ds(j * _NP + 16, 16)] = jnp.maximum(acc1, 0.0)

    dense(x_v, h_v, wl_v, bl_v, 9, 15)
    edge_stage()
    dense(agg_v, h_v, w1_v, b1_v, 15, 10)
    edge_stage()
    dense(agg_v, h_v, w2_v, b2_v, 10, 5)

    acc = jnp.zeros((16,), _f32)
    for j in range(5):
        acc = acc + h_v[pl.ds(j * _NP, 16)] * wr_v[pl.ds(j * _NP, 16)]
        acc = acc + (h_v[pl.ds(j * _NP + 16, 16)]
                     * wr_v[pl.ds(j * _NP + 16, 16)])
    outv_v[...] = wsplat(br_v, 0) + jnp.sum(acc)
    pltpu.sync_copy(outv_v.at[pl.ds(0, 1)], out_h)


@jax.jit
def _sc_call(x, edge_index, W_lift, b_lift, W1, b1, W2, b2, Wr, br):
    mesh = plsc.VectorSubcoreMesh(core_axis_name="c", subcore_axis_name="s",
                                  num_cores=2, num_subcores=16)
    f = pl.kernel(
        _sc_body,
        out_type=jax.ShapeDtypeStruct((1,), _f32),
        mesh=mesh,
        compiler_params=pltpu.CompilerParams(needs_layout_passes=False),
        scratch_types=[
            pltpu.VMEM((9 * _NP,), _f32),
            pltpu.VMEM((_EP * _F,), _i32),
            pltpu.VMEM((_EP * _F,), _i32),
            pltpu.VMEM((9 * 15,), _f32),
            pltpu.VMEM((15,), _f32),
            pltpu.VMEM((15 * 10,), _f32),
            pltpu.VMEM((10,), _f32),
            pltpu.VMEM((10 * 5,), _f32),
            pltpu.VMEM((5,), _f32),
            pltpu.VMEM((5 * _NP,), _f32),
            pltpu.VMEM((1,), _f32),
            pltpu.VMEM((_F * _NP,), _f32),
            pltpu.VMEM((_F * _NP,), _f32),
            pltpu.VMEM((_EP * _F,), _f32),
            pltpu.VMEM((_F * _NP,), _f32),
            pltpu.VMEM_SHARED((_F * _NP,), _f32),
            pltpu.VMEM_SHARED((_F * _NP,), _f32),
            pltpu.VMEM((16,), _f32),
            pltpu.SemaphoreType.DMA,
        ],
    )
    frow = jnp.arange(_F, dtype=_i32) * _NP
    srcp = jnp.concatenate([edge_index[0],
                            jnp.zeros((_EP - _E,), _i32)])
    dstp = jnp.concatenate([edge_index[1],
                            jnp.full((_EP - _E,), _NP - 1, _i32)])
    srcw = (srcp[:, None] + frow[None, :]).reshape(-1)
    dstw = (dstp[:, None] + frow[None, :]).reshape(-1)
    xT = jnp.pad(x.T, ((0, 0), (0, _NP - _N))).reshape(-1)
    wrT = jnp.pad(Wr.reshape(_N, 5).T, ((0, 0), (0, _NP - _N))).reshape(-1)
    out = f(xT, srcw, dstw, W_lift.reshape(-1),
            b_lift, W1.reshape(-1), b1, W2.reshape(-1), b2,
            wrT, br)
    return out.reshape(1, 1)


def _tc_body(src_ref, dst_ref, x_ref, wl_ref, bl_ref, w1_ref, b1_ref,
             w2_ref, b2_ref, wr_ref, br_ref, out_ref):
    f32 = jnp.float32
    nodes = jax.lax.broadcasted_iota(jnp.int32, (_N, _E), 0)
    d_oh = (dst_ref[...] == nodes).astype(f32)
    s_oh = (src_ref[...] == nodes).astype(f32)
    adj = jax.lax.dot_general(d_oh, s_oh, (((1,), (1,)), ((), ())),
                              preferred_element_type=f32)
    h = jnp.maximum(
        jnp.dot(x_ref[...], wl_ref[...], preferred_element_type=f32)
        + bl_ref[...], 0.0)
    agg = jnp.dot(adj, h, preferred_element_type=f32)
    h = jnp.maximum(
        jnp.dot(agg, w1_ref[...], preferred_element_type=f32)
        + b1_ref[...], 0.0)
    agg = jnp.dot(adj, h, preferred_element_type=f32)
    h = jnp.maximum(
        jnp.dot(agg, w2_ref[...], preferred_element_type=f32)
        + b2_ref[...], 0.0)
    out_ref[...] = jnp.sum(h * wr_ref[...])[None, None] + br_ref[...]


def _tc_call(x, edge_index, W_lift, b_lift, W1, b1, W2, b2, Wr, br):
    src = edge_index[0].reshape(1, _E)
    dst = edge_index[1].reshape(1, _E)
    out = pl.pallas_call(
        _tc_body,
        out_shape=jax.ShapeDtypeStruct((1, 1), jnp.float32),
    )(src, dst, x, W_lift, b_lift.reshape(1, -1), W1, b1.reshape(1, -1),
      W2, b2.reshape(1, -1), Wr.reshape(_N, 5), br.reshape(1, 1))
    return out


kernel = _tc_call
